# baseline (speedup 1.0000x reference)
import functools

import jax
import jax.numpy as jnp
from jax import lax
from jax.experimental import pallas as pl
from jax.experimental.pallas import tpu as pltpu
from jax.experimental.pallas import tpu_sc as plsc

_N = 100000
_D1 = 256
_D2 = 512
_NC, _NS, _L = 2, 16, 16
_NW = _NC * _NS
_CH = 120
_NCH = 26
_RW = _CH * _NCH
_TAIL = _N - _NW * _RW
_NTAILW = _TAIL // 8
_G1 = _D1 // _L
_G2 = _D2 // _L

_mesh = plsc.VectorSubcoreMesh(
    core_axis_name="c", subcore_axis_name="s",
    num_cores=_NC, num_subcores=_NS)


@functools.partial(
    pl.kernel,
    out_type=jax.ShapeDtypeStruct((_NW, 1, 2 * _D1), jnp.float32),
    mesh=_mesh,
    scratch_types=[
        pltpu.VMEM((2, _CH, _D1), jnp.float32),
        pltpu.VMEM((1, 2 * _D1), jnp.float32),
        pltpu.SemaphoreType.DMA,
        pltpu.SemaphoreType.DMA,
    ],
)
def _stats_kernel(md_hbm, out_hbm, buf, statbuf, sem0, sem1):
    wid = lax.axis_index("s") * _NC + lax.axis_index("c")
    base = wid * _RW
    sems = (sem0, sem1)

    copies = {0: pltpu.async_copy(md_hbm.at[pl.ds(base, _CH)],
                                  buf.at[0], sems[0])}
    acc = tuple(jnp.zeros((_L,), jnp.float32) for _ in range(2 * _G1))

    def make_row_body(b):
        def row_body(r, carry):
            out = list(carry)
            for c in range(_G1):
                v = buf[b, r, pl.ds(c * _L, _L)]
                out[c] = out[c] + v
                out[_G1 + c] = out[_G1 + c] + v * v
            return tuple(out)
        return row_body

    for g in range(_NCH):
        if g + 1 < _NCH:
            copies[g + 1] = pltpu.async_copy(
                md_hbm.at[pl.ds(base + (g + 1) * _CH, _CH)],
                buf.at[(g + 1) % 2], sems[(g + 1) % 2])
        copies[g].wait()
        acc = lax.fori_loop(0, _CH, make_row_body(g % 2), acc)

    has_tail = wid < _NTAILW

    @pl.when(has_tail)
    def _():
        pltpu.sync_copy(md_hbm.at[pl.ds(_NW * _RW + 8 * wid, 8)],
                        buf.at[0, pl.ds(0, 8)])

    acc = lax.fori_loop(0, jnp.where(has_tail, 8, 0),
                        make_row_body(0), acc)

    for c in range(_G1):
        statbuf[0, pl.ds(c * _L, _L)] = acc[c]
        statbuf[0, pl.ds(_D1 + c * _L, _L)] = acc[_G1 + c]
    pltpu.sync_copy(statbuf, out_hbm.at[wid])


def _encoder_body(parts_ref, x_ref, w_ref, b_ref, out_ref):
    parts = parts_ref[...]
    sums = jnp.sum(parts[:, :_D1], axis=0, keepdims=True)
    sumsq = jnp.sum(parts[:, _D1:], axis=0, keepdims=True)
    mean = sums / _N
    var = jnp.maximum((sumsq - sums * mean) / (_N - 1), 0.0)
    std = jnp.sqrt(var)
    new = (x_ref[...] - mean) / std
    new = jnp.where(std == 0.0, jnp.zeros_like(new), new)
    z = jnp.dot(new, w_ref[...], preferred_element_type=jnp.float32)
    out_ref[...] = jnp.tanh(z + b_ref[...])


_encoder = pl.pallas_call(
    _encoder_body,
    out_shape=jax.ShapeDtypeStruct((1, _D2), jnp.float32),
)


@functools.partial(
    pl.kernel,
    out_type=jax.ShapeDtypeStruct((_NW, 1, _L), jnp.float32),
    mesh=_mesh,
    scratch_types=[
        pltpu.VMEM((2, _CH, _D2), jnp.float32),
        pltpu.VMEM((_D2,), jnp.float32),
        pltpu.VMEM((1, _L), jnp.float32),
        pltpu.SemaphoreType.DMA,
        pltpu.SemaphoreType.DMA,
    ],
)
def _dist_kernel(mem_hbm, e_hbm, out_hbm, buf, e_v, min_v, sem0, sem1):
    wid = lax.axis_index("s") * _NC + lax.axis_index("c")
    base = wid * _RW
    sems = (sem0, sem1)

    pltpu.sync_copy(e_hbm, e_v)
    evecs = [e_v[pl.ds(c * _L, _L)] for c in range(_G2)]

    copies = {0: pltpu.async_copy(mem_hbm.at[pl.ds(base, _CH)],
                                  buf.at[0], sems[0])}

    iota16 = lax.iota(jnp.int32, _L)
    _dnums = lax.GatherDimensionNumbers(
        offset_dims=(), collapsed_slice_dims=(0,), start_index_map=(0,))

    def lane_total(v):
        for k in (1, 2, 4, 8):
            perm = (iota16 ^ k).reshape(_L, 1)
            v = v + lax.gather(v, perm, _dnums, slice_sizes=(1,),
                               mode=lax.GatherScatterMode.PROMISE_IN_BOUNDS)
        return v

    def make_row_body(b):
        def row_body(r, m):
            accv = jnp.abs(buf[b, r, pl.ds(0, _L)] - evecs[0])
            for c in range(1, _G2):
                accv = accv + jnp.abs(buf[b, r, pl.ds(c * _L, _L)] - evecs[c])
            return jnp.minimum(m, lane_total(accv))
        return row_body

    m = jnp.full((_L,), jnp.inf, jnp.float32)
    for g in range(_NCH):
        if g + 1 < _NCH:
            copies[g + 1] = pltpu.async_copy(
                mem_hbm.at[pl.ds(base + (g + 1) * _CH, _CH)],
                buf.at[(g + 1) % 2], sems[(g + 1) % 2])
        copies[g].wait()
        m = lax.fori_loop(0, _CH, make_row_body(g % 2), m)

    has_tail = wid < _NTAILW

    @pl.when(has_tail)
    def _():
        pltpu.sync_copy(mem_hbm.at[pl.ds(_NW * _RW + 8 * wid, 8)],
                        buf.at[0, pl.ds(0, 8)])

    m = lax.fori_loop(0, jnp.where(has_tail, 8, 0), make_row_body(0), m)

    min_v[...] = m.reshape(1, _L)
    pltpu.sync_copy(min_v, out_hbm.at[wid])


def kernel(x, memory, mem_data, W_enc, b_enc):
    parts = _stats_kernel(mem_data)
    e = _encoder(parts.reshape(_NW, 2 * _D1), x, W_enc, b_enc.reshape(1, _D2))
    mins = _dist_kernel(memory, e.reshape(_D2))
    return jnp.min(mins)
